# Initial kernel scaffold; baseline (speedup 1.0000x reference)
import jax
import jax.numpy as jnp
from jax.experimental import pallas as pl


def kernel(x0, edge_index0, edge_weight0, x1, edge_index1, edge_weight1, trainset, neg_index0, pseudo_start, pseudo_end, neg_index1, node_a, node_b, nebor_L, W01, b01, W11, b11):
    raise NotImplementedError("write your pallas kernel here")


# speedup vs baseline: 3.3165x; 3.3165x over previous
import functools

import jax
import jax.numpy as jnp
from jax import lax
from jax.experimental import pallas as pl
from jax.experimental.pallas import tpu as pltpu
from jax.experimental.pallas import tpu_sc as plsc

N = 10000
E = 320000
D = 128
TAU0 = 0.5
LN_GAMMA = -0.6931471805599453
LAMBDA1 = 1.0
T_THR = 0.1
EPS = 1e-12

_TAKE_DNUMS = lax.GatherDimensionNumbers(
    offset_dims=(), collapsed_slice_dims=(0,), start_index_map=(0,))


def _lane_bcast(v, l):
    return lax.gather(
        v, jnp.full((16, 1), l, jnp.int32), _TAKE_DNUMS, (1,),
        mode=lax.GatherScatterMode.PROMISE_IN_BOUNDS)

NC = 2
NS = 16
CHUNK = 128
EPT = E // NS
NCHUNK = -(-EPT // CHUNK)
EPT_PAD = NCHUNK * CHUNK
PAD_E = EPT_PAD * NS - E
N_PAD = 10240
ROWS_PT = N_PAD // NS

N_LC = 3072
T_ALL = 12288
NGATH = 2 * N_LC + 3 * T_ALL
GCHUNK = 11
B_PAD = NC * NS * GCHUNK * CHUNK


def _mm_body(x_ref, w_ref, b_ref, o_ref):
    o_ref[0] = (
        jnp.dot(x_ref[0], w_ref[0], preferred_element_type=jnp.float32)
        + b_ref[0]
    )


def _tc_matmul(x_st, w_st, b_st):
    rb = 1000
    return pl.pallas_call(
        _mm_body,
        grid=(2, N // rb),
        in_specs=[
            pl.BlockSpec((1, rb, D), lambda g, i: (g, i, 0)),
            pl.BlockSpec((1, D, D), lambda g, i: (g, 0, 0)),
            pl.BlockSpec((1, 1, D), lambda g, i: (g, 0, 0)),
        ],
        out_specs=pl.BlockSpec((1, rb, D), lambda g, i: (g, i, 0)),
        out_shape=jax.ShapeDtypeStruct((2, N, D), jnp.float32),
    )(x_st, w_st, b_st)


def _sc_segsum_body(s_hbm, edges_hbm, ew_hbm, h_out, edge_v, ew_v, rows_v,
                    sem, h_sh):
    c = lax.axis_index("c")
    t = lax.axis_index("s")

    def _zrow(r, carry):
        for cb in range(D // 16):
            rows_v[r, pl.ds(cb * 16, 16)] = jnp.zeros((16,), jnp.float32)
        return carry

    lax.fori_loop(0, CHUNK, _zrow, 0)
    for k in range(ROWS_PT // CHUNK):
        pltpu.sync_copy(rows_v, h_sh.at[pl.ds(t * ROWS_PT + k * CHUNK, CHUNK)])
    plsc.subcore_barrier()

    def _chunk(j, carry):
        pltpu.sync_copy(edges_hbm.at[c, t, j], edge_v)
        pltpu.sync_copy(ew_hbm.at[c, t, j], ew_v)
        pltpu.async_copy(s_hbm.at[edge_v.at[0]], rows_v, sem).wait()

        def _grp(b, c2):
            wv = ew_v[0, pl.ds(pl.multiple_of(b * 16, 16), 16)]
            for l in range(16):
                w = _lane_bcast(wv, l)
                e = b * 16 + l
                for cb in range(D // 16):
                    sl = pl.ds(cb * 16, 16)
                    rows_v[e, sl] = rows_v[e, sl] * w
            return c2

        lax.fori_loop(0, CHUNK // 16, _grp, 0)

        pltpu.sync_copy(rows_v, h_sh.at[edge_v.at[1]], add=True)
        return carry

    lax.fori_loop(0, NCHUNK, _chunk, 0)
    plsc.subcore_barrier()

    pltpu.sync_copy(h_sh.at[pl.ds(t * ROWS_PT, ROWS_PT)],
                    h_out.at[c, pl.ds(t * ROWS_PT, ROWS_PT)])


def _norm_body(h_ref, o_ref):
    h0 = h_ref[0]
    h1 = h_ref[1]
    n0 = jnp.sqrt(jnp.sum(h0 * h0, axis=1, keepdims=True))
    n1 = jnp.sqrt(jnp.sum(h1 * h1, axis=1, keepdims=True))
    o_ref[...] = jnp.concatenate([h0 / (n0 + EPS), h1 / (n1 + EPS)], axis=1)


def _tc_norm_concat(h_st):
    rb = 1000
    return pl.pallas_call(
        _norm_body,
        grid=(N // rb,),
        in_specs=[pl.BlockSpec((2, rb, D), lambda i: (0, i, 0))],
        out_specs=pl.BlockSpec((rb, 2 * D), lambda i: (i, 0)),
        out_shape=jax.ShapeDtypeStruct((N, 2 * D), jnp.float32),
    )(h_st)


def _sc_gather_body(tab_hbm, idx_hbm, out_hbm, idx_v, rows_v, sem):
    c = lax.axis_index("c")
    t = lax.axis_index("s")
    wid = t * NC + c
    pltpu.sync_copy(idx_hbm.at[wid], idx_v)

    def _j(j, carry):
        pltpu.async_copy(tab_hbm.at[idx_v.at[j]], rows_v, sem).wait()
        pltpu.sync_copy(
            rows_v, out_hbm.at[pl.ds(wid * (GCHUNK * CHUNK) + j * CHUNK, CHUNK)])
        return carry

    lax.fori_loop(0, GCHUNK, _j, 0)


@functools.lru_cache(maxsize=1)
def _sc_kernels():
    mesh = plsc.VectorSubcoreMesh(
        core_axis_name="c", subcore_axis_name="s", num_cores=NC)
    segsum = functools.partial(
        pl.kernel,
        out_type=jax.ShapeDtypeStruct((2, N_PAD, D), jnp.float32),
        mesh=mesh,
        scratch_types=[
            pltpu.VMEM((2, CHUNK), jnp.int32),
            pltpu.VMEM((1, CHUNK), jnp.float32),
            pltpu.VMEM((CHUNK, D), jnp.float32),
            pltpu.SemaphoreType.DMA,
            pltpu.VMEM_SHARED((N_PAD, D), jnp.float32),
        ],
    )(_sc_segsum_body)
    gather = functools.partial(
        pl.kernel,
        out_type=jax.ShapeDtypeStruct((B_PAD, 2 * D), jnp.float32),
        mesh=mesh,
        scratch_types=[
            pltpu.VMEM((GCHUNK, CHUNK), jnp.int32),
            pltpu.VMEM((CHUNK, 2 * D), jnp.float32),
            pltpu.SemaphoreType.DMA,
        ],
    )(_sc_gather_body)
    return segsum, gather


_BM = 512


def _lc_body(za_ref, zb_ref, nl_ref, o_ref, acc_ref):
    i = pl.program_id(0)
    za = za_ref[...]
    zb = zb_ref[...]
    za = za / (jnp.sqrt(jnp.sum(za * za, axis=1, keepdims=True)) + EPS)
    zb = zb / (jnp.sqrt(jnp.sum(zb * zb, axis=1, keepdims=True)) + EPS)
    sim = lax.dot_general(
        za, zb, (((1,), (1,)), ((), ())),
        preferred_element_type=jnp.float32) / TAU0
    col = lax.broadcasted_iota(jnp.int32, sim.shape, 1)
    row = lax.broadcasted_iota(jnp.int32, sim.shape, 0)
    pos = jnp.sum(jnp.where(col == row + i * _BM, sim, 0.0), axis=1)
    logz = jnp.log(jnp.sum(jnp.exp(sim), axis=1))
    w = jnp.exp(LN_GAMMA * nl_ref[0, 0])
    blk = jnp.sum(w * (pos - logz))

    @pl.when(i == 0)
    def _():
        acc_ref[0] = 0.0

    acc_ref[0] += blk
    o_ref[...] = jnp.full((1, 1), -acc_ref[0] / float(N_LC), jnp.float32)


def _tc_lc(za, zb, nl):
    return pl.pallas_call(
        _lc_body,
        grid=(N_LC // _BM,),
        in_specs=[
            pl.BlockSpec((_BM, 2 * D), lambda i: (i, 0)),
            pl.BlockSpec((N_LC, 2 * D), lambda i: (0, 0)),
            pl.BlockSpec((1, 1, _BM), lambda i: (i, 0, 0)),
        ],
        out_specs=pl.BlockSpec((1, 1), lambda i: (0, 0)),
        out_shape=jax.ShapeDtypeStruct((1, 1), jnp.float32),
        scratch_shapes=[pltpu.SMEM((1,), jnp.float32)],
    )(za, zb, nl)


_BB = 1024


def _bpr_body(s_ref, e_ref, n_ref, lc_ref, o_ref, acc_ref):
    i = pl.program_id(0)
    s = s_ref[...]
    e = e_ref[...]
    n = n_ref[...]
    ns = jnp.sqrt(jnp.sum(s * s, axis=1))
    ne = jnp.sqrt(jnp.sum(e * e, axis=1))
    nn = jnp.sqrt(jnp.sum(n * n, axis=1))
    pos = jnp.sum(s * e, axis=1) / (ns * ne + EPS)
    neg = jnp.sum(s * n, axis=1) / (ns * nn + EPS)
    wt = ((pos - T_THR) / (1.0 - T_THR)) ** 2
    sec = jnp.log(1.0 + jnp.exp(neg - pos))

    @pl.when(i == 0)
    def _():
        acc_ref[0] = 0.0

    acc_ref[0] += jnp.sum(wt * sec)
    o_ref[...] = jnp.full(
        (1, 1), acc_ref[0] + LAMBDA1 * lc_ref[0, 0], jnp.float32)


def _tc_bpr(s_emb, e_emb, neg, lc):
    return pl.pallas_call(
        _bpr_body,
        grid=(T_ALL // _BB,),
        in_specs=[
            pl.BlockSpec((_BB, 2 * D), lambda i: (i, 0)),
            pl.BlockSpec((_BB, 2 * D), lambda i: (i, 0)),
            pl.BlockSpec((_BB, 2 * D), lambda i: (i, 0)),
            pl.BlockSpec((1, 1), lambda i: (0, 0)),
        ],
        out_specs=pl.BlockSpec((1, 1), lambda i: (0, 0)),
        out_shape=jax.ShapeDtypeStruct((1, 1), jnp.float32),
        scratch_shapes=[pltpu.SMEM((1,), jnp.float32)],
    )(s_emb, e_emb, neg, lc)


def kernel(x0, edge_index0, edge_weight0, x1, edge_index1, edge_weight1,
           trainset, neg_index0, pseudo_start, pseudo_end, neg_index1,
           node_a, node_b, nebor_L, W01, b01, W11, b11):
    f32 = jnp.float32

    x_st = jnp.stack([x0, x1])
    w_st = jnp.stack([W01, W11])
    b_st = jnp.stack([b01, b11]).reshape(2, 1, D)
    s_st = _tc_matmul(x_st, w_st, b_st)
    s2n = s_st.reshape(2 * N, D)

    zpad = jnp.zeros((2, PAD_E), jnp.int32)
    src = jnp.concatenate(
        [jnp.stack([edge_index0[0], edge_index1[0] + N]).astype(jnp.int32),
         zpad], axis=1).reshape(2, NS, NCHUNK, CHUNK)
    dst = jnp.concatenate(
        [jnp.stack([edge_index0[1], edge_index1[1]]).astype(jnp.int32),
         zpad], axis=1).reshape(2, NS, NCHUNK, CHUNK)
    ew = jnp.concatenate(
        [jnp.stack([edge_weight0, edge_weight1]),
         jnp.zeros((2, PAD_E), f32)], axis=1).reshape(2, NS, NCHUNK, 1, CHUNK)
    edges = jnp.stack([src, dst], axis=3)

    _sc_segsum, _sc_gather = _sc_kernels()
    h_st = _sc_segsum(s2n, edges, ew)
    x_all = _tc_norm_concat(h_st)

    idx_all = jnp.concatenate([
        node_a, node_b, trainset[:, 0], pseudo_start,
        trainset[:, 1], pseudo_end, neg_index0, neg_index1,
        jnp.zeros((B_PAD - NGATH,), node_a.dtype)]).astype(jnp.int32)
    g = _sc_gather(x_all, idx_all.reshape(NC * NS, GCHUNK, CHUNK))

    za = g[0:N_LC]
    zb = g[N_LC:2 * N_LC]
    s_emb = g[2 * N_LC:2 * N_LC + T_ALL]
    e_emb = g[2 * N_LC + T_ALL:2 * N_LC + 2 * T_ALL]
    neg = g[2 * N_LC + 2 * T_ALL:2 * N_LC + 3 * T_ALL]

    nl = nebor_L.astype(f32).reshape(N_LC // _BM, 1, _BM)
    lc = _tc_lc(za, zb, nl)
    loss = _tc_bpr(s_emb, e_emb, neg, lc)

    return x_all, loss[0, 0]
